# Initial kernel scaffold; baseline (speedup 1.0000x reference)
import functools

import jax
import jax.numpy as jnp
from jax import lax
from jax.experimental import pallas as pl
from jax.experimental.pallas import tpu as pltpu
from jax.experimental.pallas import tpu_sc as plsc

_N = 10000
_E = 320000
_F = 128
_C = 40

_NP = 10240
_CP = 48
_NC = 2
_NS = 16
_NW = _NC * _NS
_CH = 128
_EPW = 10240
_EP = _EPW * _NW
_NCHUNK = _EPW // _CH
_RPT = _NP // _NS
_BR = 1024


def _mesh():
    return plsc.VectorSubcoreMesh(core_axis_name="c", subcore_axis_name="s")


def _sc_degree(dst3):
    @functools.partial(
        pl.kernel,
        out_type=jax.ShapeDtypeStruct((_NC, _NP, 16), jnp.float32),
        mesh=_mesh(),
        scratch_types=[
            pltpu.VMEM((_NCHUNK, _CH), jnp.int32),
            pltpu.VMEM((_CH, 16), jnp.float32),
            pltpu.VMEM_SHARED((_NP, 16), jnp.float32),
            pltpu.SemaphoreType.DMA,
        ],
    )
    def deg_kernel(dst_hbm, out_hbm, dst_v, val_v, acc_sh, sem):
        c = lax.axis_index("c")
        s = lax.axis_index("s")
        wid = c * _NS + s
        pltpu.sync_copy(dst_hbm.at[wid], dst_v)
        zero = jnp.zeros((16,), jnp.float32)

        def _fill_zero(r, _):
            val_v[r, :] = zero
            return 0

        lax.fori_loop(0, _CH, _fill_zero, 0)
        for t in range(_RPT // _CH):
            pltpu.sync_copy(val_v, acc_sh.at[pl.ds(s * _RPT + t * _CH, _CH)])
        one = jnp.ones((16,), jnp.float32)

        def _fill_one(r, _):
            val_v[r, :] = one
            return 0

        lax.fori_loop(0, _CH, _fill_one, 0)
        plsc.subcore_barrier()

        def _chunk(j, _):
            pltpu.sync_copy(val_v, acc_sh.at[dst_v.at[j]], add=True)
            return 0

        lax.fori_loop(0, _NCHUNK, _chunk, 0)
        plsc.subcore_barrier()
        pltpu.sync_copy(acc_sh.at[pl.ds(s * _RPT, _RPT)],
                        out_hbm.at[c, pl.ds(s * _RPT, _RPT)])

    return deg_kernel(dst3)


def _sc_scatter(g, src3, dst3, d):
    @functools.partial(
        pl.kernel,
        out_type=jax.ShapeDtypeStruct((_NC, _NP, d), jnp.float32),
        mesh=_mesh(),
        scratch_types=[
            pltpu.VMEM((_NCHUNK, _CH), jnp.int32),
            pltpu.VMEM((_NCHUNK, _CH), jnp.int32),
            pltpu.VMEM((_CH, d), jnp.float32),
            pltpu.VMEM_SHARED((_NP, d), jnp.float32),
            pltpu.SemaphoreType.DMA,
        ],
    )
    def scat_kernel(g_hbm, src_hbm, dst_hbm, out_hbm, src_v, dst_v, rows_v,
                    acc_sh, sem):
        c = lax.axis_index("c")
        s = lax.axis_index("s")
        wid = c * _NS + s
        pltpu.sync_copy(src_hbm.at[wid], src_v)
        pltpu.sync_copy(dst_hbm.at[wid], dst_v)
        zero = jnp.zeros((16,), jnp.float32)

        def _fill_zero(r, _):
            for k in range(d // 16):
                rows_v[r, pl.ds(k * 16, 16)] = zero
            return 0

        lax.fori_loop(0, _CH, _fill_zero, 0)
        for t in range(_RPT // _CH):
            pltpu.sync_copy(rows_v, acc_sh.at[pl.ds(s * _RPT + t * _CH, _CH)])
        plsc.subcore_barrier()

        def _chunk(j, _):
            pltpu.async_copy(g_hbm.at[src_v.at[j]], rows_v, sem).wait()
            pltpu.sync_copy(rows_v, acc_sh.at[dst_v.at[j]], add=True)
            return 0

        lax.fori_loop(0, _NCHUNK, _chunk, 0)
        plsc.subcore_barrier()
        pltpu.sync_copy(acc_sh.at[pl.ds(s * _RPT, _RPT)],
                        out_hbm.at[c, pl.ds(s * _RPT, _RPT)])

    return scat_kernel(g, src3, dst3)


def _dinv_block(dp0, dp1):
    cnt = dp0[:, 0:1] + dp1[:, 0:1]
    return lax.rsqrt(cnt + 1.0)


def _tc1_body(dp0, dp1, x_r, w_r, g_r):
    dinv = _dinv_block(dp0, dp1)
    g_r[...] = dinv * jnp.dot(x_r[...], w_r[...],
                              preferred_element_type=jnp.float32)


def _tc2_body(dp0, dp1, p0, p1, g1, b1r, w2, hid_r, g2_r):
    dinv = _dinv_block(dp0, dp1)
    o1 = dinv * (p0[...] + p1[...] + g1[...]) + b1r[...]
    hid = jnp.maximum(o1, 0.0)
    hid_r[...] = hid
    g2_r[...] = dinv * jnp.dot(hid, w2[...], preferred_element_type=jnp.float32)


def _tc3_body(dp0, dp1, q0, q1, g2, b2r, o2_r, lp_r):
    dinv = _dinv_block(dp0, dp1)
    o2 = dinv * (q0[...] + q1[...] + g2[...]) + b2r[...]
    col = lax.broadcasted_iota(jnp.int32, (_BR, _CP), 1)
    valid = col < _C
    masked = jnp.where(valid, o2, -jnp.inf)
    m = jnp.max(masked, axis=1, keepdims=True)
    ex = jnp.where(valid, jnp.exp(o2 - m), 0.0)
    lse = jnp.log(jnp.sum(ex, axis=1, keepdims=True))
    o2_r[...] = o2
    lp_r[...] = o2 - (m + lse)


def _row_spec(w):
    return pl.BlockSpec((_BR, w), lambda i: (i, 0))


def _full_spec(h, w):
    return pl.BlockSpec((h, w), lambda i: (0, 0))


_GRID = (_NP // _BR,)


def _tc1(dp0, dp1, xp, w1):
    return pl.pallas_call(
        _tc1_body,
        grid=_GRID,
        in_specs=[_row_spec(16), _row_spec(16), _row_spec(_F), _full_spec(_F, _F)],
        out_specs=_row_spec(_F),
        out_shape=jax.ShapeDtypeStruct((_NP, _F), jnp.float32),
    )(dp0, dp1, xp, w1)


def _tc2(dp0, dp1, p0, p1, g1, b1r, w2p):
    return pl.pallas_call(
        _tc2_body,
        grid=_GRID,
        in_specs=[_row_spec(16), _row_spec(16), _row_spec(_F), _row_spec(_F),
                  _row_spec(_F), _full_spec(1, _F), _full_spec(_F, _CP)],
        out_specs=[_row_spec(_F), _row_spec(_CP)],
        out_shape=[jax.ShapeDtypeStruct((_NP, _F), jnp.float32),
                   jax.ShapeDtypeStruct((_NP, _CP), jnp.float32)],
    )(dp0, dp1, p0, p1, g1, b1r, w2p)


def _tc3(dp0, dp1, q0, q1, g2, b2r):
    return pl.pallas_call(
        _tc3_body,
        grid=_GRID,
        in_specs=[_row_spec(16), _row_spec(16), _row_spec(_CP), _row_spec(_CP),
                  _row_spec(_CP), _full_spec(1, _CP)],
        out_specs=[_row_spec(_CP), _row_spec(_CP)],
        out_shape=[jax.ShapeDtypeStruct((_NP, _CP), jnp.float32),
                   jax.ShapeDtypeStruct((_NP, _CP), jnp.float32)],
    )(dp0, dp1, q0, q1, g2, b2r)


def kernel(x, edge_index, W1, b1, W2, b2):
    src = edge_index[0].astype(jnp.int32)
    dst = edge_index[1].astype(jnp.int32)
    pad_idx = jnp.full((_EP - _E,), _N, jnp.int32)
    src3 = jnp.concatenate([src, pad_idx]).reshape(_NW, _NCHUNK, _CH)
    dst3 = jnp.concatenate([dst, pad_idx]).reshape(_NW, _NCHUNK, _CH)
    xp = jnp.zeros((_NP, _F), jnp.float32).at[:_N].set(x)
    w2p = jnp.zeros((_F, _CP), jnp.float32).at[:, :_C].set(W2)
    b1r = b1.reshape(1, _F)
    b2r = jnp.zeros((1, _CP), jnp.float32).at[0, :_C].set(b2)

    degp = _sc_degree(dst3)
    dp0, dp1 = degp[0], degp[1]
    g1 = _tc1(dp0, dp1, xp, W1)
    p = _sc_scatter(g1, src3, dst3, _F)
    hid, g2 = _tc2(dp0, dp1, p[0], p[1], g1, b1r, w2p)
    q = _sc_scatter(g2, src3, dst3, _CP)
    o2, lp = _tc3(dp0, dp1, q[0], q[1], g2, b2r)
    return (hid[:_N], o2[:_N, :_C], lp[:_N, :_C])


# speedup vs baseline: 9.7304x; 9.7304x over previous
import functools

import jax
import jax.numpy as jnp
from jax import lax
from jax.experimental import pallas as pl
from jax.experimental.pallas import tpu as pltpu
from jax.experimental.pallas import tpu_sc as plsc

_N = 10000
_E = 320000
_F = 128
_C = 40

_NP = 10240
_CP = 48
_NC = 2
_NS = 16
_NW = _NC * _NS
_CH = 128
_EPW = 10240
_EP = _EPW * _NW
_NCHUNK = _EPW // _CH
_RPT = _NP // _NS
_BR = 1024


def _mesh():
    return plsc.VectorSubcoreMesh(core_axis_name="c", subcore_axis_name="s")


def _sc_degree(dst3):
    @functools.partial(
        pl.kernel,
        out_type=jax.ShapeDtypeStruct((_NC, _NP, 16), jnp.float32),
        mesh=_mesh(),
        scratch_types=[
            pltpu.VMEM((_NCHUNK, _CH), jnp.int32),
            pltpu.VMEM((_CH, 16), jnp.float32),
            pltpu.VMEM_SHARED((_NP, 16), jnp.float32),
            pltpu.SemaphoreType.DMA,
        ],
    )
    def deg_kernel(dst_hbm, out_hbm, dst_v, val_v, acc_sh, sem):
        c = lax.axis_index("c")
        s = lax.axis_index("s")
        wid = c * _NS + s
        pltpu.sync_copy(dst_hbm.at[wid], dst_v)
        zero = jnp.zeros((16,), jnp.float32)

        def _fill_zero(r, _):
            val_v[r, :] = zero
            return 0

        lax.fori_loop(0, _CH, _fill_zero, 0)
        for t in range(_RPT // _CH):
            pltpu.sync_copy(val_v, acc_sh.at[pl.ds(s * _RPT + t * _CH, _CH)])
        one = jnp.ones((16,), jnp.float32)

        def _fill_one(r, _):
            val_v[r, :] = one
            return 0

        lax.fori_loop(0, _CH, _fill_one, 0)
        plsc.subcore_barrier()

        def _chunk(j, _):
            pltpu.sync_copy(val_v, acc_sh.at[dst_v.at[j]], add=True)
            return 0

        lax.fori_loop(0, _NCHUNK, _chunk, 0)
        plsc.subcore_barrier()
        pltpu.sync_copy(acc_sh.at[pl.ds(s * _RPT, _RPT)],
                        out_hbm.at[c, pl.ds(s * _RPT, _RPT)])

    return deg_kernel(dst3)


def _sc_scatter(g, src3, dst3, d):
    @functools.partial(
        pl.kernel,
        out_type=jax.ShapeDtypeStruct((_NC, _NP, d), jnp.float32),
        mesh=_mesh(),
        scratch_types=[
            pltpu.VMEM((_NCHUNK, _CH), jnp.int32),
            pltpu.VMEM((_NCHUNK, _CH), jnp.int32),
            pltpu.VMEM((_CH, d), jnp.float32),
            pltpu.VMEM_SHARED((_NP, d), jnp.float32),
            pltpu.SemaphoreType.DMA,
        ],
    )
    def scat_kernel(g_hbm, src_hbm, dst_hbm, out_hbm, src_v, dst_v, rows_v,
                    acc_sh, sem):
        c = lax.axis_index("c")
        s = lax.axis_index("s")
        wid = c * _NS + s
        pltpu.sync_copy(src_hbm.at[wid], src_v)
        pltpu.sync_copy(dst_hbm.at[wid], dst_v)
        zero = jnp.zeros((16,), jnp.float32)

        def _fill_zero(r, _):
            for k in range(d // 16):
                rows_v[r, pl.ds(k * 16, 16)] = zero
            return 0

        lax.fori_loop(0, _CH, _fill_zero, 0)
        for t in range(_RPT // _CH):
            pltpu.sync_copy(rows_v, acc_sh.at[pl.ds(s * _RPT + t * _CH, _CH)])
        plsc.subcore_barrier()

        def _chunk(j, _):
            pltpu.async_copy(g_hbm.at[src_v.at[j]], rows_v, sem).wait()
            pltpu.sync_copy(rows_v, acc_sh.at[dst_v.at[j]], add=True)
            return 0

        lax.fori_loop(0, _NCHUNK, _chunk, 0)
        plsc.subcore_barrier()
        pltpu.sync_copy(acc_sh.at[pl.ds(s * _RPT, _RPT)],
                        out_hbm.at[c, pl.ds(s * _RPT, _RPT)])

    return scat_kernel(g, src3, dst3)


def _dinv_block(dp0, dp1):
    cnt = dp0[:, 0:1] + dp1[:, 0:1]
    return lax.rsqrt(cnt + 1.0)


def _tc1_body(dp0, dp1, x_r, w_r, g_r):
    dinv = _dinv_block(dp0, dp1)
    g_r[...] = dinv * jnp.dot(x_r[...], w_r[...],
                              preferred_element_type=jnp.float32)


def _tc2_body(dp0, dp1, p0, p1, g1, b1r, hid_r, gh_r):
    dinv = _dinv_block(dp0, dp1)
    o1 = dinv * (p0[...] + p1[...] + g1[...]) + b1r[...]
    hid = jnp.maximum(o1, 0.0)
    hid_r[...] = hid
    gh_r[...] = dinv * hid


def _tc3_body(dp0, dp1, q0, q1, gh, w2, b2r, o2_r, lp_r):
    dinv = _dinv_block(dp0, dp1)
    u = dinv * (q0[...] + q1[...] + gh[...])
    o2 = jnp.dot(u, w2[...], preferred_element_type=jnp.float32) + b2r[...]
    col = lax.broadcasted_iota(jnp.int32, (_BR, _CP), 1)
    valid = col < _C
    masked = jnp.where(valid, o2, -jnp.inf)
    m = jnp.max(masked, axis=1, keepdims=True)
    ex = jnp.where(valid, jnp.exp(o2 - m), 0.0)
    lse = jnp.log(jnp.sum(ex, axis=1, keepdims=True))
    o2_r[...] = o2
    lp_r[...] = o2 - (m + lse)


def _row_spec(w):
    return pl.BlockSpec((_BR, w), lambda i: (i, 0))


def _full_spec(h, w):
    return pl.BlockSpec((h, w), lambda i: (0, 0))


_GRID = (_NP // _BR,)


def _tc1(dp0, dp1, xp, w1):
    return pl.pallas_call(
        _tc1_body,
        grid=_GRID,
        in_specs=[_row_spec(16), _row_spec(16), _row_spec(_F), _full_spec(_F, _F)],
        out_specs=_row_spec(_F),
        out_shape=jax.ShapeDtypeStruct((_NP, _F), jnp.float32),
    )(dp0, dp1, xp, w1)


def _tc2(dp0, dp1, p0, p1, g1, b1r):
    return pl.pallas_call(
        _tc2_body,
        grid=_GRID,
        in_specs=[_row_spec(16), _row_spec(16), _row_spec(_F), _row_spec(_F),
                  _row_spec(_F), _full_spec(1, _F)],
        out_specs=[_row_spec(_F), _row_spec(_F)],
        out_shape=[jax.ShapeDtypeStruct((_NP, _F), jnp.float32),
                   jax.ShapeDtypeStruct((_NP, _F), jnp.float32)],
    )(dp0, dp1, p0, p1, g1, b1r)


def _tc3(dp0, dp1, q0, q1, gh, w2p, b2r):
    return pl.pallas_call(
        _tc3_body,
        grid=_GRID,
        in_specs=[_row_spec(16), _row_spec(16), _row_spec(_F), _row_spec(_F),
                  _row_spec(_F), _full_spec(_F, _CP), _full_spec(1, _CP)],
        out_specs=[_row_spec(_CP), _row_spec(_CP)],
        out_shape=[jax.ShapeDtypeStruct((_NP, _CP), jnp.float32),
                   jax.ShapeDtypeStruct((_NP, _CP), jnp.float32)],
    )(dp0, dp1, q0, q1, gh, w2p, b2r)


def kernel(x, edge_index, W1, b1, W2, b2):
    src = edge_index[0].astype(jnp.int32)
    dst = edge_index[1].astype(jnp.int32)
    pad_idx = jnp.full((_EP - _E,), _N, jnp.int32)
    src3 = jnp.concatenate([src, pad_idx]).reshape(_NW, _NCHUNK, _CH)
    dst3 = jnp.concatenate([dst, pad_idx]).reshape(_NW, _NCHUNK, _CH)
    xp = jnp.zeros((_NP, _F), jnp.float32).at[:_N].set(x)
    w2p = jnp.zeros((_F, _CP), jnp.float32).at[:, :_C].set(W2)
    b1r = b1.reshape(1, _F)
    b2r = jnp.zeros((1, _CP), jnp.float32).at[0, :_C].set(b2)

    degp = _sc_degree(dst3)
    dp0, dp1 = degp[0], degp[1]
    g1 = _tc1(dp0, dp1, xp, W1)
    p = _sc_scatter(g1, src3, dst3, _F)
    hid, gh = _tc2(dp0, dp1, p[0], p[1], g1, b1r)
    q = _sc_scatter(gh, src3, dst3, _F)
    o2, lp = _tc3(dp0, dp1, q[0], q[1], gh, w2p, b2r)
    return (hid[:_N], o2[:_N, :_C], lp[:_N, :_C])


# speedup vs baseline: 10.5414x; 1.0833x over previous
import functools

import jax
import jax.numpy as jnp
from jax import lax
from jax.experimental import pallas as pl
from jax.experimental.pallas import tpu as pltpu
from jax.experimental.pallas import tpu_sc as plsc

_N = 10000
_E = 320000
_F = 128
_C = 40

_NP = 10240
_CP = 48
_NC = 2
_NS = 16
_NW = _NC * _NS
_CH = 128
_EPW = 10240
_EP = _EPW * _NW
_NCHUNK = _EPW // _CH
_RPT = _NP // _NS
_BR = 1024


def _mesh():
    return plsc.VectorSubcoreMesh(core_axis_name="c", subcore_axis_name="s")


def _sc_degree(dst3):
    @functools.partial(
        pl.kernel,
        out_type=jax.ShapeDtypeStruct((_NC, _NP, 16), jnp.float32),
        mesh=_mesh(),
        scratch_types=[
            pltpu.VMEM((_NCHUNK, _CH), jnp.int32),
            pltpu.VMEM((_CH, 16), jnp.float32),
            pltpu.VMEM_SHARED((_NP, 16), jnp.float32),
            pltpu.SemaphoreType.DMA,
        ],
    )
    def deg_kernel(dst_hbm, out_hbm, dst_v, val_v, acc_sh, sem):
        c = lax.axis_index("c")
        s = lax.axis_index("s")
        wid = c * _NS + s
        pltpu.sync_copy(dst_hbm.at[wid], dst_v)
        zero = jnp.zeros((16,), jnp.float32)

        def _fill_zero(r, _):
            val_v[r, :] = zero
            return 0

        lax.fori_loop(0, _CH, _fill_zero, 0)
        for t in range(_RPT // _CH):
            pltpu.sync_copy(val_v, acc_sh.at[pl.ds(s * _RPT + t * _CH, _CH)])
        one = jnp.ones((16,), jnp.float32)

        def _fill_one(r, _):
            val_v[r, :] = one
            return 0

        lax.fori_loop(0, _CH, _fill_one, 0)
        plsc.subcore_barrier()

        def _chunk(j, _):
            pltpu.sync_copy(val_v, acc_sh.at[dst_v.at[j]], add=True)
            return 0

        lax.fori_loop(0, _NCHUNK, _chunk, 0)
        plsc.subcore_barrier()
        pltpu.sync_copy(acc_sh.at[pl.ds(s * _RPT, _RPT)],
                        out_hbm.at[c, pl.ds(s * _RPT, _RPT)])

    return deg_kernel(dst3)


_SCH = 128
_SNCH = _EPW // _SCH
_NB = 2
_BLKCH = 16


def _sc_scatter(g, src3, dst3, d):
    @functools.partial(
        pl.kernel,
        out_type=jax.ShapeDtypeStruct((_NC, _NP, d), jnp.float32),
        mesh=_mesh(),
        scratch_types=[
            pltpu.VMEM((_BLKCH, _SCH), jnp.int32),
            pltpu.VMEM((_BLKCH, _SCH), jnp.int32),
            pltpu.VMEM((_SCH, d), jnp.float32),
            pltpu.VMEM((_SCH, d), jnp.float32),
            pltpu.VMEM_SHARED((_NP, d), jnp.float32),
        ] + [pltpu.SemaphoreType.DMA] * 4,
    )
    def scat_kernel(g_hbm, src_hbm, dst_hbm, out_hbm, src_v, dst_v,
                    rows_a, rows_b, acc_sh, gs0, gs1, ss0, ss1):
        rows = (rows_a, rows_b)
        gsem = (gs0, gs1)
        ssem = (ss0, ss1)
        c = lax.axis_index("c")
        s = lax.axis_index("s")
        wid = c * _NS + s
        zero = jnp.zeros((16,), jnp.float32)

        def _fill_zero(r, _):
            for k in range(d // 16):
                rows_a[r, pl.ds(k * 16, 16)] = zero
                rows_b[r, pl.ds(k * 16, 16)] = zero
            return 0

        lax.fori_loop(0, _SCH, _fill_zero, 0)
        for t in range(_RPT // (2 * _SCH)):
            pltpu.sync_copy(rows_a,
                            acc_sh.at[pl.ds(s * _RPT + (2 * t) * _SCH, _SCH)])
            pltpu.sync_copy(rows_b,
                            acc_sh.at[pl.ds(s * _RPT + (2 * t + 1) * _SCH, _SCH)])
        plsc.subcore_barrier()

        def _gather(j, b):
            pltpu.async_copy(g_hbm.at[src_v.at[j]], rows[b], gsem[b])

        def _wait_gather(j, b):
            pltpu.make_async_copy(g_hbm.at[src_v.at[j]], rows[b],
                                  gsem[b]).wait()

        def _scatter(j, b):
            pltpu.async_copy(rows[b], acc_sh.at[dst_v.at[j]],
                             ssem[b], add=True)

        def _wait_scatter(j, b):
            pltpu.make_async_copy(rows[b], acc_sh.at[dst_v.at[j]],
                                  ssem[b]).wait()

        for blk in range(_SNCH // _BLKCH):
            base = blk * _BLKCH
            pltpu.sync_copy(src_hbm.at[wid, pl.ds(base, _BLKCH)], src_v)
            pltpu.sync_copy(dst_hbm.at[wid, pl.ds(base, _BLKCH)], dst_v)
            _gather(0, 0)
            _wait_gather(0, 0)
            _scatter(0, 0)
            _gather(1, 1)
            _wait_gather(1, 1)
            _scatter(1, 1)
            _wait_scatter(0, 0)
            _gather(2, 0)

            def _pair(t, _):
                for b in range(2):
                    j = 2 * t + b
                    _wait_gather(j, b)
                    _scatter(j, b)
                    _wait_scatter(j - 1, 1 - b)
                    _gather(j + 1, 1 - b)
                return 0

            lax.fori_loop(1, _BLKCH // 2 - 1, _pair, 0)
            j0 = _BLKCH - 2
            _wait_gather(j0, 0)
            _scatter(j0, 0)
            _wait_scatter(j0 - 1, 1)
            _gather(j0 + 1, 1)
            _wait_gather(j0 + 1, 1)
            _scatter(j0 + 1, 1)
            _wait_scatter(j0, 0)
            _wait_scatter(j0 + 1, 1)
        plsc.subcore_barrier()
        pltpu.sync_copy(acc_sh.at[pl.ds(s * _RPT, _RPT)],
                        out_hbm.at[c, pl.ds(s * _RPT, _RPT)])

    return scat_kernel(g, src3, dst3)


def _dinv_block(dp0, dp1):
    cnt = dp0[:, 0:1] + dp1[:, 0:1]
    return lax.rsqrt(cnt + 1.0)


def _tc1_body(dp0, dp1, x_r, w_r, g_r):
    dinv = _dinv_block(dp0, dp1)
    g_r[...] = dinv * jnp.dot(x_r[...], w_r[...],
                              preferred_element_type=jnp.float32)


def _tc2_body(dp0, dp1, p0, p1, g1, b1r, hid_r, gh_r):
    dinv = _dinv_block(dp0, dp1)
    o1 = dinv * (p0[...] + p1[...] + g1[...]) + b1r[...]
    hid = jnp.maximum(o1, 0.0)
    hid_r[...] = hid
    gh_r[...] = dinv * hid


def _tc3_body(dp0, dp1, q0, q1, gh, w2, b2r, o2_r, lp_r):
    dinv = _dinv_block(dp0, dp1)
    u = dinv * (q0[...] + q1[...] + gh[...])
    o2 = jnp.dot(u, w2[...], preferred_element_type=jnp.float32) + b2r[...]
    col = lax.broadcasted_iota(jnp.int32, (_BR, _CP), 1)
    valid = col < _C
    masked = jnp.where(valid, o2, -jnp.inf)
    m = jnp.max(masked, axis=1, keepdims=True)
    ex = jnp.where(valid, jnp.exp(o2 - m), 0.0)
    lse = jnp.log(jnp.sum(ex, axis=1, keepdims=True))
    o2_r[...] = o2
    lp_r[...] = o2 - (m + lse)


def _row_spec(w):
    return pl.BlockSpec((_BR, w), lambda i: (i, 0))


def _full_spec(h, w):
    return pl.BlockSpec((h, w), lambda i: (0, 0))


_GRID = (_NP // _BR,)


def _tc1(dp0, dp1, xp, w1):
    return pl.pallas_call(
        _tc1_body,
        grid=_GRID,
        in_specs=[_row_spec(16), _row_spec(16), _row_spec(_F), _full_spec(_F, _F)],
        out_specs=_row_spec(_F),
        out_shape=jax.ShapeDtypeStruct((_NP, _F), jnp.float32),
    )(dp0, dp1, xp, w1)


def _tc2(dp0, dp1, p0, p1, g1, b1r):
    return pl.pallas_call(
        _tc2_body,
        grid=_GRID,
        in_specs=[_row_spec(16), _row_spec(16), _row_spec(_F), _row_spec(_F),
                  _row_spec(_F), _full_spec(1, _F)],
        out_specs=[_row_spec(_F), _row_spec(_F)],
        out_shape=[jax.ShapeDtypeStruct((_NP, _F), jnp.float32),
                   jax.ShapeDtypeStruct((_NP, _F), jnp.float32)],
    )(dp0, dp1, p0, p1, g1, b1r)


def _tc3(dp0, dp1, q0, q1, gh, w2p, b2r):
    return pl.pallas_call(
        _tc3_body,
        grid=_GRID,
        in_specs=[_row_spec(16), _row_spec(16), _row_spec(_F), _row_spec(_F),
                  _row_spec(_F), _full_spec(_F, _CP), _full_spec(1, _CP)],
        out_specs=[_row_spec(_CP), _row_spec(_CP)],
        out_shape=[jax.ShapeDtypeStruct((_NP, _CP), jnp.float32),
                   jax.ShapeDtypeStruct((_NP, _CP), jnp.float32)],
    )(dp0, dp1, q0, q1, gh, w2p, b2r)


def kernel(x, edge_index, W1, b1, W2, b2):
    src = edge_index[0].astype(jnp.int32)
    dst = edge_index[1].astype(jnp.int32)
    pad_idx = jnp.full((_EP - _E,), _N, jnp.int32)
    srcp = jnp.concatenate([src, pad_idx])
    dstp = jnp.concatenate([dst, pad_idx])
    srcs = srcp.reshape(_NW, _SNCH, _SCH)
    dsts = dstp.reshape(_NW, _SNCH, _SCH)
    dstd = dstp.reshape(_NW, _NCHUNK, _CH)
    xp = jnp.zeros((_NP, _F), jnp.float32).at[:_N].set(x)
    w2p = jnp.zeros((_F, _CP), jnp.float32).at[:, :_C].set(W2)
    b1r = b1.reshape(1, _F)
    b2r = jnp.zeros((1, _CP), jnp.float32).at[0, :_C].set(b2)

    degp = _sc_degree(dstd)
    dp0, dp1 = degp[0], degp[1]
    g1 = _tc1(dp0, dp1, xp, W1)
    p = _sc_scatter(g1, srcs, dsts, _F)
    hid, gh = _tc2(dp0, dp1, p[0], p[1], g1, b1r)
    q = _sc_scatter(gh, srcs, dsts, _F)
    o2, lp = _tc3(dp0, dp1, q[0], q[1], gh, w2p, b2r)
    return (hid[:_N], o2[:_N, :_C], lp[:_N, :_C])
